# baseline (speedup 1.0000x reference)
import functools

import jax
import jax.numpy as jnp
from jax import lax
from jax.experimental import pallas as pl
from jax.experimental.pallas import tpu as pltpu
from jax.experimental.pallas import tpu_sc as plsc

_LEN = 2048
_C = 128
_B = 32
_NC = 2
_NS = 16
_NW = _NC * _NS
_PB = 128
_NPB = _LEN // _PB
_BH = _B // 2


def _pe_table():
    i = jnp.arange(_LEN, dtype=jnp.float32)[:, None]
    j = jnp.arange(_C, dtype=jnp.float32)[None, :]
    val = i / jnp.power(10000.0, j / float(_C))
    even = (jnp.arange(_C)[None, :] % 2) == 0
    return jnp.where(even, jnp.sin(val), jnp.cos(val))


@functools.partial(
    pl.kernel,
    out_type=jax.ShapeDtypeStruct((_B, _LEN, _C), jnp.float32),
    mesh=plsc.VectorSubcoreMesh(core_axis_name="c", subcore_axis_name="s"),
    scratch_types=[
        pltpu.VMEM((_BH, _PB), jnp.int32),
        pltpu.VMEM((_PB, _C), jnp.float32),
        pltpu.SemaphoreType.DMA,
    ],
)
def _embed_sc(x_hbm, w_hbm, pe_hbm, out_hbm, idx_v, buf_v, sem):
    wid = lax.axis_index("s") * _NC + lax.axis_index("c")
    j = wid // 2
    h = wid % 2
    p0 = j * _PB
    b0 = h * _BH
    pltpu.sync_copy(x_hbm.at[pl.ds(b0, _BH), pl.ds(p0, _PB)], idx_v)

    def body(b, carry):
        pltpu.sync_copy(pe_hbm.at[pl.ds(p0, _PB)], buf_v)
        pltpu.async_copy(w_hbm.at[idx_v.at[b]], buf_v, sem, add=True).wait()
        pltpu.sync_copy(buf_v, out_hbm.at[b0 + b, pl.ds(p0, _PB)])
        return carry

    lax.fori_loop(0, _BH, body, 0)


def kernel(x, W):
    pe = _pe_table()
    return _embed_sc(x.astype(jnp.int32), W, pe)


# speedup vs baseline: 1.4358x; 1.4358x over previous
import functools

import jax
import jax.numpy as jnp
from jax import lax
from jax.experimental import pallas as pl
from jax.experimental.pallas import tpu as pltpu
from jax.experimental.pallas import tpu_sc as plsc

_LEN = 2048
_C = 128
_B = 32
_NC = 2
_NS = 16
_NW = _NC * _NS
_PB = 128
_NPB = _LEN // _PB
_BH = _B // 2


def _pe_table():
    i = jnp.arange(_LEN, dtype=jnp.float32)[:, None]
    j = jnp.arange(_C, dtype=jnp.float32)[None, :]
    val = i / jnp.power(10000.0, j / float(_C))
    even = (jnp.arange(_C)[None, :] % 2) == 0
    return jnp.where(even, jnp.sin(val), jnp.cos(val))


@functools.partial(
    pl.kernel,
    out_type=jax.ShapeDtypeStruct((_B, _LEN, _C), jnp.float32),
    mesh=plsc.VectorSubcoreMesh(core_axis_name="c", subcore_axis_name="s"),
    scratch_types=[
        pltpu.VMEM((_BH, _PB), jnp.int32),
        pltpu.VMEM((2, _PB, _C), jnp.float32),
        pltpu.VMEM_SHARED((_NS, _PB, _C), jnp.float32),
        pltpu.SemaphoreType.DMA,
        pltpu.SemaphoreType.DMA,
        pltpu.SemaphoreType.DMA,
        pltpu.SemaphoreType.DMA,
        pltpu.SemaphoreType.DMA,
        pltpu.SemaphoreType.DMA,
    ],
)
def _embed_sc(x_hbm, w_hbm, pe_hbm, out_hbm, idx_v, buf_v, pe_sh,
              sem_s0, sem_s1, sem_g0, sem_g1, sem_o0, sem_o1):
    c = lax.axis_index("c")
    s = lax.axis_index("s")
    p0 = s * _PB
    b0 = c * _BH
    pltpu.sync_copy(x_hbm.at[pl.ds(b0, _BH), pl.ds(p0, _PB)], idx_v)
    pltpu.sync_copy(pe_hbm.at[pl.ds(p0, _PB)], buf_v.at[0])
    pltpu.sync_copy(buf_v.at[0], pe_sh.at[s])
    pe_slot = pe_sh.at[s]

    sems_s = (sem_s0, sem_s1)
    sems_g = (sem_g0, sem_g1)
    sems_o = (sem_o0, sem_o1)
    seed = [None, None]
    outw = [None, None]

    seed[0] = pltpu.async_copy(pe_slot, buf_v.at[0], sems_s[0])
    for b in range(_BH):
        sl = b % 2
        seed[sl].wait()
        gat = pltpu.async_copy(w_hbm.at[idx_v.at[b]], buf_v.at[sl],
                               sems_g[sl], add=True)
        if b + 1 < _BH:
            ns = 1 - sl
            if b >= 1:
                outw[ns].wait()
            seed[ns] = pltpu.async_copy(pe_slot, buf_v.at[ns], sems_s[ns])
        gat.wait()
        outw[sl] = pltpu.async_copy(buf_v.at[sl],
                                    out_hbm.at[b0 + b, pl.ds(p0, _PB)],
                                    sems_o[sl])
    outw[0].wait()
    outw[1].wait()


def kernel(x, W):
    pe = _pe_table()
    return _embed_sc(x.astype(jnp.int32), W, pe)


# speedup vs baseline: 1.5527x; 1.0815x over previous
import functools

import jax
import jax.numpy as jnp
from jax import lax
from jax.experimental import pallas as pl
from jax.experimental.pallas import tpu as pltpu
from jax.experimental.pallas import tpu_sc as plsc

_LEN = 2048
_C = 128
_B = 32
_NC = 2
_NS = 16
_NW = _NC * _NS
_PB = 128
_NPB = _LEN // _PB
_BH = _B // 2


def _pe_table():
    i = jnp.arange(_LEN, dtype=jnp.float32)[:, None]
    j = jnp.arange(_C, dtype=jnp.float32)[None, :]
    val = i / jnp.power(10000.0, j / float(_C))
    even = (jnp.arange(_C)[None, :] % 2) == 0
    return jnp.where(even, jnp.sin(val), jnp.cos(val))


@functools.partial(
    pl.kernel,
    out_type=jax.ShapeDtypeStruct((_B, _LEN, _C), jnp.float32),
    mesh=plsc.VectorSubcoreMesh(core_axis_name="c", subcore_axis_name="s"),
    scratch_types=[
        pltpu.VMEM((_BH, _PB), jnp.int32),
        pltpu.VMEM((4, _PB, _C), jnp.float32),
        pltpu.VMEM_SHARED((_NS, _PB, _C), jnp.float32),
        pltpu.SemaphoreType.DMA((4,)),
        pltpu.SemaphoreType.DMA((4,)),
        pltpu.SemaphoreType.DMA((4,)),
    ],
)
def _embed_sc(x_hbm, w_hbm, pe_hbm, out_hbm, idx_v, buf_v, pe_sh,
              sems_s, sems_g, sems_o):
    c = lax.axis_index("c")
    s = lax.axis_index("s")
    p0 = s * _PB
    b0 = c * _BH
    pltpu.sync_copy(x_hbm.at[pl.ds(b0, _BH), pl.ds(p0, _PB)], idx_v)
    pltpu.sync_copy(pe_hbm.at[pl.ds(p0, _PB)], buf_v.at[0])
    pltpu.sync_copy(buf_v.at[0], pe_sh.at[s])
    pe_slot = pe_sh.at[s]

    seed = [None] * 4
    gat = [None] * 4
    outw = [None] * 4

    seed[0] = pltpu.async_copy(pe_slot, buf_v.at[0], sems_s.at[0])
    seed[1] = pltpu.async_copy(pe_slot, buf_v.at[1], sems_s.at[1])
    for t in range(_BH + 1):
        bs = t + 2
        if bs < _BH:
            sl = bs % 4
            if bs >= 4:
                outw[sl].wait()
            seed[sl] = pltpu.async_copy(pe_slot, buf_v.at[sl], sems_s.at[sl])
        if t < _BH:
            sl = t % 4
            seed[sl].wait()
            gat[sl] = pltpu.async_copy(w_hbm.at[idx_v.at[t]], buf_v.at[sl],
                                       sems_g.at[sl], add=True)
        bw = t - 1
        if bw >= 0:
            sl = bw % 4
            gat[sl].wait()
            outw[sl] = pltpu.async_copy(buf_v.at[sl],
                                        out_hbm.at[b0 + bw, pl.ds(p0, _PB)],
                                        sems_o.at[sl])
    for b in range(_BH - 4, _BH):
        outw[b % 4].wait()


def kernel(x, W):
    pe = _pe_table()
    return _embed_sc(x.astype(jnp.int32), W, pe)


# speedup vs baseline: 1.6260x; 1.0472x over previous
import functools

import jax
import jax.numpy as jnp
from jax import lax
from jax.experimental import pallas as pl
from jax.experimental.pallas import tpu as pltpu
from jax.experimental.pallas import tpu_sc as plsc

_LEN = 2048
_C = 128
_B = 32
_NC = 2
_NS = 16
_PB = 128
_BH = _B // 2
_R = 6
_GLAG = 4


def _pe_table():
    i = jnp.arange(_LEN, dtype=jnp.float32)[:, None]
    j = jnp.arange(_C, dtype=jnp.float32)[None, :]
    val = i / jnp.power(10000.0, j / float(_C))
    even = (jnp.arange(_C)[None, :] % 2) == 0
    return jnp.where(even, jnp.sin(val), jnp.cos(val))


@functools.partial(
    pl.kernel,
    out_type=jax.ShapeDtypeStruct((_B, _LEN, _C), jnp.float32),
    mesh=plsc.VectorSubcoreMesh(core_axis_name="c", subcore_axis_name="s"),
    scratch_types=[
        pltpu.VMEM((_BH, _PB), jnp.int32),
        pltpu.VMEM((_R, _PB, _C), jnp.float32),
        pltpu.VMEM_SHARED((_NS, _PB, _C), jnp.float32),
        pltpu.SemaphoreType.DMA((_R,)),
        pltpu.SemaphoreType.DMA((_R,)),
        pltpu.SemaphoreType.DMA((_R,)),
    ],
)
def _embed_sc(x_hbm, w_hbm, pe_hbm, out_hbm, idx_v, buf_v, pe_sh,
              sems_s, sems_g, sems_o):
    c = lax.axis_index("c")
    s = lax.axis_index("s")
    p0 = s * _PB
    b0 = c * _BH
    pltpu.sync_copy(x_hbm.at[pl.ds(b0, _BH), pl.ds(p0, _PB)], idx_v)
    pltpu.sync_copy(pe_hbm.at[pl.ds(p0, _PB)], buf_v.at[0])
    pltpu.sync_copy(buf_v.at[0], pe_sh.at[s])
    pe_slot = pe_sh.at[s]

    seed = [None] * _R
    gat = [None] * _R
    outw = [None] * _R

    seed[0] = pltpu.async_copy(pe_slot, buf_v.at[0], sems_s.at[0])
    for t in range(_BH + _GLAG + 1):
        bw = t - _GLAG
        if 0 <= bw < _BH:
            sl = bw % _R
            gat[sl].wait()
            outw[sl] = pltpu.async_copy(
                buf_v.at[sl], out_hbm.at[b0 + bw, pl.ds(p0, _PB)],
                sems_o.at[sl])
        bs = t + 1
        if bs < _BH:
            sl = bs % _R
            if bs >= _R:
                outw[sl].wait()
            seed[sl] = pltpu.async_copy(pe_slot, buf_v.at[sl], sems_s.at[sl])
        if t < _BH:
            sl = t % _R
            seed[sl].wait()
            gat[sl] = pltpu.async_copy(
                w_hbm.at[idx_v.at[t]], buf_v.at[sl], sems_g.at[sl], add=True)
    for b in range(_BH - _R, _BH):
        outw[b % _R].wait()


def kernel(x, W):
    pe = _pe_table()
    return _embed_sc(x.astype(jnp.int32), W, pe)
